# Initial kernel scaffold; baseline (speedup 1.0000x reference)
import jax
import jax.numpy as jnp
from jax.experimental import pallas as pl


def kernel(x, w1s, b1s, w2s, b2s, wf1, bf1, wf2, bf2, wf3, bf3):
    raise NotImplementedError("write your pallas kernel here")


# speedup vs baseline: 13.3470x; 13.3470x over previous
import numpy as np

import jax
import jax.numpy as jnp
from jax.experimental import pallas as pl
from jax.experimental.pallas import tpu as pltpu

_BF16 = jnp.bfloat16
_F32 = jnp.float32


def _cnn_body(x_ref, w1e_ref, w1o_ref, b1_ref, w2e_ref, w2o_ref, b2_ref,
              wf1_ref, bf1_ref, wf2_ref, bf2_ref, wf3_ref, bf3_ref,
              out_ref, p1_ref, p2_ref):
    def dot(a, b):
        return jnp.dot(a, b, preferred_element_type=_F32)

    w1e = w1e_ref[...]
    w1o = w1o_ref[...]
    b1 = b1_ref[...]
    for jj in range(14):
        x0 = x_ref[pl.ds(jj * 192, 480), :]
        x1 = x_ref[pl.ds(jj * 192 + 96, 480), :]
        y = jnp.maximum(jnp.maximum(dot(w1e, x0), dot(w1o, x0)),
                        jnp.maximum(dot(w1e, x1), dot(w1o, x1)))
        y = jnp.maximum(y + b1, 0.0)
        p1_ref[pl.ds(jj * 96, 96), :] = y.astype(_BF16)

    w2e = w2e_ref[...]
    w2o = w2o_ref[...]
    b2 = b2_ref[...]
    for jj in range(5):
        x0 = p1_ref[pl.ds(jj * 192, 480), :]
        x1 = p1_ref[pl.ds(jj * 192 + 96, 480), :]
        y = jnp.maximum(jnp.maximum(dot(w2e, x0), dot(w2o, x0)),
                        jnp.maximum(dot(w2e, x1), dot(w2o, x1)))
        y = jnp.maximum(y + b2, 0.0)
        p2_ref[pl.ds(jj * 80, 80), :] = y.astype(_BF16)

    h1 = jnp.maximum(dot(wf1_ref[...], p2_ref[...]) + bf1_ref[...], 0.0)
    h2 = jnp.maximum(dot(wf2_ref[...], h1.astype(_BF16)) + bf2_ref[...], 0.0)
    logits = dot(wf3_ref[...], h2.astype(_BF16)) + bf3_ref[...]

    m = jnp.max(logits, axis=0, keepdims=True)
    z = logits - m
    s = jnp.sum(jnp.exp(z), axis=0, keepdims=True)
    out_ref[...] = z - jnp.log(s)


def _toeplitz_onehot(n_u, n_w, parity):
    e = np.zeros((n_u, 5, n_w), np.float32)
    for u in range(n_u):
        for kw in range(5):
            e[u, kw, 2 * u + parity + kw] = 1.0
    return jnp.asarray(e)


def kernel(x, w1s, b1s, w2s, b2s, wf1, bf1, wf2, bf2, wf3, bf3):
    B = x.shape[0]
    TB = 256 if B >= 256 else B
    Bp = ((B + TB - 1) // TB) * TB

    xr = jnp.transpose(x, (2, 1, 3, 0)).reshape(3072, B).astype(_BF16)
    if Bp != B:
        xr = jnp.pad(xr, ((0, 0), (0, Bp - B)))

    w1 = w1s.reshape(6, 3, 5, 5)
    w1mats = []
    for p in (0, 1):
        e = _toeplitz_onehot(14, 32, p)
        m = jnp.einsum('ochk,ukw->ouhcw', w1, e).reshape(84, 480)
        w1mats.append(jnp.pad(m, ((0, 12), (0, 0))).astype(_BF16))
    b1r = jnp.pad(jnp.repeat(b1s, 14), (0, 12)).reshape(96, 1)

    w2 = w2s.reshape(16, 6, 5, 5)
    w2mats = []
    for p in (0, 1):
        e = _toeplitz_onehot(5, 14, p)
        m = jnp.einsum('ochk,ukw->ouhcw', w2, e).reshape(80, 5, 84)
        m = jnp.pad(m, ((0, 0), (0, 0), (0, 12))).reshape(80, 480)
        w2mats.append(m.astype(_BF16))
    b2r = jnp.repeat(b2s, 5).reshape(80, 1)

    wf1f = jnp.transpose(wf1, (1, 0, 2)).reshape(120, 400).astype(_BF16)
    wf2c = wf2.astype(_BF16)
    wf3c = wf3.astype(_BF16)

    def vfull(a):
        return pl.BlockSpec(a.shape, lambda i: (0,) * a.ndim)

    in_specs = [
        pl.BlockSpec((3072, TB), lambda i: (0, i)),
        vfull(w1mats[0]), vfull(w1mats[1]), vfull(b1r),
        vfull(w2mats[0]), vfull(w2mats[1]), vfull(b2r),
        vfull(wf1f), vfull(bf1), vfull(wf2c), vfull(bf2),
        vfull(wf3c), vfull(bf3),
    ]

    macs = Bp * (6 * 28 * 28 * 75 + 16 * 10 * 10 * 150
                 + 400 * 120 + 120 * 84 + 84 * 10)
    cost = pl.CostEstimate(flops=2 * macs,
                           transcendentals=11 * Bp,
                           bytes_accessed=2 * Bp * 3072 + 4 * Bp * 10 + 70000)

    out = pl.pallas_call(
        _cnn_body,
        out_shape=jax.ShapeDtypeStruct((10, Bp), _F32),
        grid=(Bp // TB,),
        in_specs=in_specs,
        out_specs=pl.BlockSpec((10, TB), lambda i: (0, i)),
        scratch_shapes=[pltpu.VMEM((14 * 96, TB), _BF16),
                        pltpu.VMEM((5 * 80, TB), _BF16)],
        compiler_params=pltpu.CompilerParams(
            dimension_semantics=("parallel",),
            vmem_limit_bytes=48 * 1024 * 1024),
        cost_estimate=cost,
    )(xr, w1mats[0], w1mats[1], b1r, w2mats[0], w2mats[1], b2r,
      wf1f, bf1, wf2c, bf2, wf3c, bf3)

    return jnp.transpose(out)[:B]


# speedup vs baseline: 13.3718x; 1.0019x over previous
import numpy as np

import jax
import jax.numpy as jnp
from jax.experimental import pallas as pl
from jax.experimental.pallas import tpu as pltpu

_BF16 = jnp.bfloat16
_F32 = jnp.float32


def _cnn_body(x_ref, w1e_ref, w1o_ref, b1_ref, w2e_ref, w2o_ref, b2_ref,
              wf1_ref, bf1_ref, wf2_ref, bf2_ref, wf3_ref, bf3_ref,
              out_ref, p1_ref, p2_ref):
    def dot(a, b):
        return jnp.dot(a, b, preferred_element_type=_F32)

    w1e = w1e_ref[...]
    w1o = w1o_ref[...]
    b1 = b1_ref[...]
    for jj in range(14):
        x0 = x_ref[pl.ds(jj * 192, 480), :]
        x1 = x_ref[pl.ds(jj * 192 + 96, 480), :]
        y = jnp.maximum(jnp.maximum(dot(w1e, x0), dot(w1o, x0)),
                        jnp.maximum(dot(w1e, x1), dot(w1o, x1)))
        y = jnp.maximum(y + b1, 0.0)
        p1_ref[pl.ds(jj * 96, 96), :] = y.astype(_BF16)

    w2e = w2e_ref[...]
    w2o = w2o_ref[...]
    b2 = b2_ref[...]
    for jj in range(5):
        x0 = p1_ref[pl.ds(jj * 192, 480), :]
        x1 = p1_ref[pl.ds(jj * 192 + 96, 480), :]
        y = jnp.maximum(jnp.maximum(dot(w2e, x0), dot(w2o, x0)),
                        jnp.maximum(dot(w2e, x1), dot(w2o, x1)))
        y = jnp.maximum(y + b2, 0.0)
        p2_ref[pl.ds(jj * 80, 80), :] = y.astype(_BF16)

    h1 = jnp.maximum(dot(wf1_ref[...], p2_ref[...]) + bf1_ref[...], 0.0)
    h2 = jnp.maximum(dot(wf2_ref[...], h1.astype(_BF16)) + bf2_ref[...], 0.0)
    logits = dot(wf3_ref[...], h2.astype(_BF16)) + bf3_ref[...]

    m = jnp.max(logits, axis=0, keepdims=True)
    z = logits - m
    s = jnp.sum(jnp.exp(z), axis=0, keepdims=True)
    out_ref[...] = z - jnp.log(s)


def _toeplitz_onehot(n_u, n_w, parity):
    e = np.zeros((n_u, 5, n_w), np.float32)
    for u in range(n_u):
        for kw in range(5):
            e[u, kw, 2 * u + parity + kw] = 1.0
    return jnp.asarray(e)


def kernel(x, w1s, b1s, w2s, b2s, wf1, bf1, wf2, bf2, wf3, bf3):
    B = x.shape[0]
    TB = 256 if B >= 256 else B
    Bp = ((B + TB - 1) // TB) * TB

    xr = jnp.transpose(x.astype(_BF16), (2, 1, 3, 0)).reshape(3072, B)
    if Bp != B:
        xr = jnp.pad(xr, ((0, 0), (0, Bp - B)))

    w1 = w1s.reshape(6, 3, 5, 5)
    w1mats = []
    for p in (0, 1):
        e = _toeplitz_onehot(14, 32, p)
        m = jnp.einsum('ochk,ukw->ouhcw', w1, e).reshape(84, 480)
        w1mats.append(jnp.pad(m, ((0, 12), (0, 0))).astype(_BF16))
    b1r = jnp.pad(jnp.repeat(b1s, 14), (0, 12)).reshape(96, 1)

    w2 = w2s.reshape(16, 6, 5, 5)
    w2mats = []
    for p in (0, 1):
        e = _toeplitz_onehot(5, 14, p)
        m = jnp.einsum('ochk,ukw->ouhcw', w2, e).reshape(80, 5, 84)
        m = jnp.pad(m, ((0, 0), (0, 0), (0, 12))).reshape(80, 480)
        w2mats.append(m.astype(_BF16))
    b2r = jnp.repeat(b2s, 5).reshape(80, 1)

    wf1f = jnp.transpose(wf1, (1, 0, 2)).reshape(120, 400).astype(_BF16)
    wf2c = wf2.astype(_BF16)
    wf3c = wf3.astype(_BF16)

    def vfull(a):
        return pl.BlockSpec(a.shape, lambda i: (0,) * a.ndim)

    in_specs = [
        pl.BlockSpec((3072, TB), lambda i: (0, i)),
        vfull(w1mats[0]), vfull(w1mats[1]), vfull(b1r),
        vfull(w2mats[0]), vfull(w2mats[1]), vfull(b2r),
        vfull(wf1f), vfull(bf1), vfull(wf2c), vfull(bf2),
        vfull(wf3c), vfull(bf3),
    ]

    macs = Bp * (6 * 28 * 28 * 75 + 16 * 10 * 10 * 150
                 + 400 * 120 + 120 * 84 + 84 * 10)
    cost = pl.CostEstimate(flops=2 * macs,
                           transcendentals=11 * Bp,
                           bytes_accessed=2 * Bp * 3072 + 4 * Bp * 10 + 70000)

    out = pl.pallas_call(
        _cnn_body,
        out_shape=jax.ShapeDtypeStruct((10, Bp), _F32),
        grid=(Bp // TB,),
        in_specs=in_specs,
        out_specs=pl.BlockSpec((10, TB), lambda i: (0, i)),
        scratch_shapes=[pltpu.VMEM((14 * 96, TB), _BF16),
                        pltpu.VMEM((5 * 80, TB), _BF16)],
        compiler_params=pltpu.CompilerParams(
            dimension_semantics=("parallel",),
            vmem_limit_bytes=48 * 1024 * 1024),
        cost_estimate=cost,
    )(xr, w1mats[0], w1mats[1], b1r, w2mats[0], w2mats[1], b2r,
      wf1f, bf1, wf2c, bf2, wf3c, bf3)

    return jnp.transpose(out)[:B]
